# Initial kernel scaffold; baseline (speedup 1.0000x reference)
import jax
import jax.numpy as jnp
from jax.experimental import pallas as pl


def kernel(x2d, voxel_indices, img_indices, dist_to_cam):
    raise NotImplementedError("write your pallas kernel here")


# speedup vs baseline: 8.9605x; 8.9605x over previous
import functools

import jax
import jax.numpy as jnp
from jax import lax
from jax.experimental import pallas as pl
from jax.experimental.pallas import tpu as pltpu
from jax.experimental.pallas import tpu_sc as plsc

_C = 128
_H = 120
_W = 160
_HW = _H * _W
_SCENE = (60, 36, 60)
_TOTAL = _SCENE[0] * _SCENE[1] * _SCENE[2]

_L = 16
_NW = 32
_K = 64
_NG = _K // _L
_CG = _C // _L
_NCHUNK = _TOTAL // _K
_ITERS = -(-_NCHUNK // _NW)


def _sc_droi(table, winner, coords):
    n_pts = coords.shape[0]
    mesh = plsc.VectorSubcoreMesh(core_axis_name="c", subcore_axis_name="s")

    @functools.partial(
        pl.kernel,
        mesh=mesh,
        compiler_params=pltpu.CompilerParams(needs_layout_passes=False),
        out_type=jax.ShapeDtypeStruct((_TOTAL, _C), jnp.float32),
        scratch_types=[
            pltpu.VMEM((_K,), jnp.int32),
            pltpu.VMEM((_K,), jnp.int32),
            pltpu.VMEM((_K,), jnp.int32),
            pltpu.VMEM((_K,), jnp.int32),
            pltpu.VMEM((_K,), jnp.int32),
            pltpu.VMEM((_K,), jnp.int32),
            pltpu.VMEM((_K,), jnp.int32),
            pltpu.VMEM((_K,), jnp.float32),
            pltpu.VMEM((_K,), jnp.float32),
            pltpu.VMEM((_K,), jnp.float32),
            pltpu.VMEM((_K,), jnp.float32),
            pltpu.VMEM((_K, _C), jnp.float32),
            pltpu.VMEM((_K, _C), jnp.float32),
            pltpu.VMEM((_K, _C), jnp.float32),
            pltpu.VMEM((_K, _C), jnp.float32),
            pltpu.VMEM((_K, _C), jnp.float32),
            pltpu.SemaphoreType.DMA,
        ],
    )
    def k(table_hbm, winner_hbm, crd_hbm, out_hbm,
          win_vm, widx_vm, crd_vm,
          p0_vm, p1_vm, p2_vm, p3_vm,
          w0_vm, w1_vm, w2_vm, w3_vm,
          g0_vm, g1_vm, g2_vm, g3_vm,
          out_vm, sem):
        wid = lax.axis_index("s") * 2 + lax.axis_index("c")
        iota = lax.iota(jnp.int32, _L)
        zeros = jnp.zeros((_L,), jnp.int32)

        def chunk_body(i, carry):
            chunk = wid + _NW * i

            @pl.when(chunk < _NCHUNK)
            def _():
                base = pl.multiple_of(chunk * _K, _K)
                pltpu.sync_copy(winner_hbm.at[pl.ds(base, _K)], win_vm)
                for g in range(_NG):
                    w = win_vm[pl.ds(g * _L, _L)]
                    pad = lax.rem(base + g * _L + iota, n_pts)
                    widx_vm[pl.ds(g * _L, _L)] = jnp.where(w >= 0, w, pad)
                pltpu.async_copy(crd_hbm.at[widx_vm], crd_vm, sem).wait()
                for g in range(_NG):
                    sl = pl.ds(g * _L, _L)
                    w = win_vm[sl]
                    crd = crd_vm[sl]
                    yf = lax.shift_right_logical(crd, 9).astype(jnp.float32)
                    xf = jnp.bitwise_and(crd, 511).astype(jnp.float32)
                    y0 = (yf - 2.0) * 0.25
                    x0 = (xf - 2.0) * 0.25
                    valid = ((y0 >= -1.0) & (y0 <= float(_H))
                             & (x0 >= -1.0) & (x0 <= float(_W)))
                    keep = valid & (w >= 0)
                    y = jnp.maximum(y0, 0.0)
                    x = jnp.maximum(x0, 0.0)
                    yl = y.astype(jnp.int32)
                    xl = x.astype(jnp.int32)
                    ly = jnp.where(yl >= _H - 1, 0.0, y - yl.astype(jnp.float32))
                    lx = jnp.where(xl >= _W - 1, 0.0, x - xl.astype(jnp.float32))
                    yl = jnp.minimum(yl, _H - 1)
                    xl = jnp.minimum(xl, _W - 1)
                    yh = jnp.minimum(yl + 1, _H - 1)
                    xh = jnp.minimum(xl + 1, _W - 1)
                    scale = jnp.where(keep, 1.0, 0.0)
                    hy = (1.0 - ly) * scale
                    lys = ly * scale
                    hx = 1.0 - lx
                    p0_vm[sl] = yl * _W + xl
                    p1_vm[sl] = yl * _W + xh
                    p2_vm[sl] = yh * _W + xl
                    p3_vm[sl] = yh * _W + xh
                    w0_vm[sl] = hy * hx
                    w1_vm[sl] = hy * lx
                    w2_vm[sl] = lys * hx
                    w3_vm[sl] = lys * lx
                c0 = pltpu.async_copy(table_hbm.at[p0_vm], g0_vm, sem)
                c1 = pltpu.async_copy(table_hbm.at[p1_vm], g1_vm, sem)
                c2 = pltpu.async_copy(table_hbm.at[p2_vm], g2_vm, sem)
                c3 = pltpu.async_copy(table_hbm.at[p3_vm], g3_vm, sem)
                c0.wait()
                c1.wait()
                c2.wait()
                c3.wait()
                for g in range(_NG):

                    def vbody(j, carry2, g=g):
                        jj = zeros + (g * _L) + j
                        b0 = plsc.load_gather(w0_vm, [jj])
                        b1 = plsc.load_gather(w1_vm, [jj])
                        b2 = plsc.load_gather(w2_vm, [jj])
                        b3 = plsc.load_gather(w3_vm, [jj])
                        v = g * _L + j
                        for cg in range(_CG):
                            cs = pl.ds(cg * _L, _L)
                            out_vm[v, cs] = (b0 * g0_vm[v, cs]
                                             + b1 * g1_vm[v, cs]
                                             + b2 * g2_vm[v, cs]
                                             + b3 * g3_vm[v, cs])
                        return carry2

                    lax.fori_loop(0, _L, vbody, 0)
                pltpu.sync_copy(out_vm, out_hbm.at[pl.ds(base, _K), :])

            return carry

        lax.fori_loop(0, _ITERS, chunk_body, 0)

    return k(table, winner, coords)


def kernel(x2d, voxel_indices, img_indices, dist_to_cam):
    del dist_to_cam
    table = jnp.transpose(x2d, (1, 2, 0)).reshape(_HW, _C)
    n = voxel_indices.shape[0]
    flat = (voxel_indices[:, 0] * (_SCENE[1] * _SCENE[2])
            + voxel_indices[:, 1] * _SCENE[2]
            + voxel_indices[:, 2]).astype(jnp.int32)
    winner = jnp.full((_TOTAL,), -1, jnp.int32).at[flat].max(
        jnp.arange(n, dtype=jnp.int32))
    img = img_indices.astype(jnp.int32)
    coords = img[:, 0] * 512 + img[:, 1]
    out = _sc_droi(table, winner, coords)
    return jnp.transpose(out).reshape(_C, *_SCENE)


# speedup vs baseline: 11.0687x; 1.2353x over previous
import functools

import jax
import jax.numpy as jnp
from jax import lax
from jax.experimental import pallas as pl
from jax.experimental.pallas import tpu as pltpu
from jax.experimental.pallas import tpu_sc as plsc

_C = 128
_H = 120
_W = 160
_HW = _H * _W
_SCENE = (60, 36, 60)
_TOTAL = _SCENE[0] * _SCENE[1] * _SCENE[2]

_L = 16
_NW = 32
_K = 64
_NG = _K // _L
_CG = _C // _L

_GROUPS = _TOTAL // _L
_GRP_LO = _GROUPS // _NW
_GRP_EXTRA = _GROUPS % _NW
_LEN_MAX = (_GRP_LO + 1) * _L
_CAP = 4096


def _sc_droi(table, winner, coords):
    n_pts = coords.shape[0]
    mesh = plsc.VectorSubcoreMesh(core_axis_name="c", subcore_axis_name="s")

    @functools.partial(
        pl.kernel,
        mesh=mesh,
        compiler_params=pltpu.CompilerParams(needs_layout_passes=False),
        out_type=jax.ShapeDtypeStruct((_TOTAL + _K, _C), jnp.float32),
        scratch_types=[
            pltpu.VMEM((_LEN_MAX,), jnp.int32),
            pltpu.VMEM((_CAP,), jnp.int32),
            pltpu.VMEM((_CAP,), jnp.int32),
            pltpu.VMEM((_K,), jnp.int32),
            pltpu.VMEM((_K,), jnp.int32),
            pltpu.VMEM((_K,), jnp.int32),
            pltpu.VMEM((_K,), jnp.int32),
            pltpu.VMEM((_K,), jnp.int32),
            pltpu.VMEM((_K,), jnp.int32),
            pltpu.VMEM((_K,), jnp.int32),
            pltpu.VMEM((_K,), jnp.float32),
            pltpu.VMEM((_K,), jnp.float32),
            pltpu.VMEM((_K,), jnp.float32),
            pltpu.VMEM((_K,), jnp.float32),
            pltpu.VMEM((_K, _C), jnp.float32),
            pltpu.VMEM((_K, _C), jnp.float32),
            pltpu.VMEM((_K, _C), jnp.float32),
            pltpu.VMEM((_K, _C), jnp.float32),
            pltpu.VMEM((_K, _C), jnp.float32),
            pltpu.SemaphoreType.DMA,
        ],
    )
    def k(table_hbm, winner_hbm, crd_hbm, out_hbm,
          wall_vm, cvox_vm, cwin_vm, voxc_vm, winc_vm, crd_vm,
          p0_vm, p1_vm, p2_vm, p3_vm,
          w0_vm, w1_vm, w2_vm, w3_vm,
          g0_vm, g1_vm, g2_vm, g3_vm,
          out_vm, sem):
        wid = lax.axis_index("s") * 2 + lax.axis_index("c")
        iota = lax.iota(jnp.int32, _L)
        zeros = jnp.zeros((_L,), jnp.int32)

        ngrp = jnp.where(wid < _GRP_EXTRA, _GRP_LO + 1, _GRP_LO)
        start = (wid * _GRP_LO + jnp.minimum(wid, _GRP_EXTRA)) * _L
        start = pl.multiple_of(start, _L)

        for g in range(_CAP // _L):
            sl = pl.ds(g * _L, _L)
            cvox_vm[sl] = _TOTAL + jnp.bitwise_and(g + iota, _K - 1)
            cwin_vm[sl] = lax.rem(start + g * _L + iota, n_pts)

        pltpu.sync_copy(winner_hbm.at[pl.ds(start, _LEN_MAX)], wall_vm)

        def abody(g, off):
            w = wall_vm[pl.ds(g * _L, _L)]
            mask = w >= 0
            sl = pl.ds(off, _L)
            plsc.store_compressed(cvox_vm.at[sl], start + g * _L + iota,
                                  mask=mask)
            plsc.store_compressed(cwin_vm.at[sl], w, mask=mask)
            return off + jnp.sum(mask.astype(jnp.int32))

        nc = lax.fori_loop(0, ngrp, abody, 0)
        nloop = lax.shift_right_logical(nc + (_K - 1), 6)

        def bbody(it, carry):
            cb = it * _K
            for g in range(_NG):
                sl = pl.ds(g * _L, _L)
                voxc_vm[sl] = cvox_vm[pl.ds(cb + g * _L, _L)]
                winc_vm[sl] = cwin_vm[pl.ds(cb + g * _L, _L)]
            pltpu.async_copy(crd_hbm.at[winc_vm], crd_vm, sem).wait()
            for g in range(_NG):
                sl = pl.ds(g * _L, _L)
                real = (cb + g * _L + iota) < nc
                crd = crd_vm[sl]
                yf = lax.shift_right_logical(crd, 9).astype(jnp.float32)
                xf = jnp.bitwise_and(crd, 511).astype(jnp.float32)
                y0 = (yf - 2.0) * 0.25
                x0 = (xf - 2.0) * 0.25
                valid = ((y0 >= -1.0) & (y0 <= float(_H))
                         & (x0 >= -1.0) & (x0 <= float(_W)))
                keep = valid & real
                y = jnp.maximum(y0, 0.0)
                x = jnp.maximum(x0, 0.0)
                yl = y.astype(jnp.int32)
                xl = x.astype(jnp.int32)
                ly = jnp.where(yl >= _H - 1, 0.0, y - yl.astype(jnp.float32))
                lx = jnp.where(xl >= _W - 1, 0.0, x - xl.astype(jnp.float32))
                yl = jnp.minimum(yl, _H - 1)
                xl = jnp.minimum(xl, _W - 1)
                yh = jnp.minimum(yl + 1, _H - 1)
                xh = jnp.minimum(xl + 1, _W - 1)
                scale = jnp.where(keep, 1.0, 0.0)
                hy = (1.0 - ly) * scale
                lys = ly * scale
                hx = 1.0 - lx
                p0_vm[sl] = yl * _W + xl
                p1_vm[sl] = yl * _W + xh
                p2_vm[sl] = yh * _W + xl
                p3_vm[sl] = yh * _W + xh
                w0_vm[sl] = hy * hx
                w1_vm[sl] = hy * lx
                w2_vm[sl] = lys * hx
                w3_vm[sl] = lys * lx
            c0 = pltpu.async_copy(table_hbm.at[p0_vm], g0_vm, sem)
            c1 = pltpu.async_copy(table_hbm.at[p1_vm], g1_vm, sem)
            c2 = pltpu.async_copy(table_hbm.at[p2_vm], g2_vm, sem)
            c3 = pltpu.async_copy(table_hbm.at[p3_vm], g3_vm, sem)
            c0.wait()
            c1.wait()
            c2.wait()
            c3.wait()
            for g in range(_NG):

                def vbody(j, carry2, g=g):
                    jj = zeros + (g * _L) + j
                    b0 = plsc.load_gather(w0_vm, [jj])
                    b1 = plsc.load_gather(w1_vm, [jj])
                    b2 = plsc.load_gather(w2_vm, [jj])
                    b3 = plsc.load_gather(w3_vm, [jj])
                    v = g * _L + j
                    for cg in range(_CG):
                        cs = pl.ds(cg * _L, _L)
                        out_vm[v, cs] = (b0 * g0_vm[v, cs]
                                         + b1 * g1_vm[v, cs]
                                         + b2 * g2_vm[v, cs]
                                         + b3 * g3_vm[v, cs])
                    return carry2

                lax.fori_loop(0, _L, vbody, 0)
            pltpu.async_copy(out_vm, out_hbm.at[voxc_vm], sem).wait()
            return carry

        lax.fori_loop(0, nloop, bbody, 0)

    return k(table, winner, coords)


def kernel(x2d, voxel_indices, img_indices, dist_to_cam):
    del dist_to_cam
    table = jnp.transpose(x2d, (1, 2, 0)).reshape(_HW, _C)
    n = voxel_indices.shape[0]
    flat = (voxel_indices[:, 0] * (_SCENE[1] * _SCENE[2])
            + voxel_indices[:, 1] * _SCENE[2]
            + voxel_indices[:, 2]).astype(jnp.int32)
    winner = jnp.full((_TOTAL,), -1, jnp.int32).at[flat].max(
        jnp.arange(n, dtype=jnp.int32))
    img = img_indices.astype(jnp.int32)
    coords = img[:, 0] * 512 + img[:, 1]
    out = _sc_droi(table, winner, coords)
    res = jnp.where(winner[None, :] >= 0, jnp.transpose(out[:_TOTAL]), 0.0)
    return res.reshape(_C, *_SCENE)


# speedup vs baseline: 12.7487x; 1.1518x over previous
import functools

import jax
import jax.numpy as jnp
from jax import lax
from jax.experimental import pallas as pl
from jax.experimental.pallas import tpu as pltpu
from jax.experimental.pallas import tpu_sc as plsc

_C = 128
_H = 120
_W = 160
_HW = _H * _W
_SCENE = (60, 36, 60)
_TOTAL = _SCENE[0] * _SCENE[1] * _SCENE[2]

_L = 16
_NW = 32
_K = 64
_NG = _K // _L
_CG = _C // _L
_CB = 128

_GROUPS = _TOTAL // _L
_GRP_LO = _GROUPS // _NW
_GRP_EXTRA = _GROUPS % _NW
_LEN_MAX = (_GRP_LO + 1) * _L
_CAP = 4096


def _sc_droi(table, winner, coords):
    n_pts = coords.shape[0]
    mesh = plsc.VectorSubcoreMesh(core_axis_name="c", subcore_axis_name="s")

    @functools.partial(
        pl.kernel,
        mesh=mesh,
        compiler_params=pltpu.CompilerParams(needs_layout_passes=False),
        out_type=jax.ShapeDtypeStruct((_TOTAL + _K, _C), jnp.float32),
        scratch_types=[
            pltpu.VMEM((_CAP,), jnp.int32),
            pltpu.VMEM((_CAP,), jnp.int32),
            pltpu.VMEM((_CAP,), jnp.int32),
            pltpu.VMEM((2, _K), jnp.int32),
            pltpu.VMEM((2, 4, _K), jnp.int32),
            pltpu.VMEM((2, 4, _K), jnp.float32),
            pltpu.VMEM((2, _K, _C), jnp.float32),
            pltpu.VMEM((2, _K, _C), jnp.float32),
            pltpu.VMEM((2, _K, _C), jnp.float32),
            pltpu.VMEM((2, _K, _C), jnp.float32),
            pltpu.VMEM((2, _K, _C), jnp.float32),
            pltpu.SemaphoreType.DMA,
            pltpu.SemaphoreType.DMA,
            pltpu.SemaphoreType.DMA,
            pltpu.SemaphoreType.DMA,
            pltpu.SemaphoreType.DMA,
        ],
    )
    def k(table_hbm, winner_hbm, crd_hbm, out_hbm,
          scr_vm, cvox_vm, cwin_vm, voxc_vm, p_vm, w_vm,
          g0_vm, g1_vm, g2_vm, g3_vm, out_vm,
          csem, gsem0, gsem1, ssem0, ssem1):
        wid = lax.axis_index("s") * 2 + lax.axis_index("c")
        iota = lax.iota(jnp.int32, _L)
        zeros = jnp.zeros((_L,), jnp.int32)
        gsem = (gsem0, gsem1)
        ssem = (ssem0, ssem1)
        gbufs = (g0_vm, g1_vm, g2_vm, g3_vm)

        ngrp = jnp.where(wid < _GRP_EXTRA, _GRP_LO + 1, _GRP_LO)
        start = (wid * _GRP_LO + jnp.minimum(wid, _GRP_EXTRA)) * _L
        start = pl.multiple_of(start, _L)

        def fbody(g, carry):
            sl = pl.ds(pl.multiple_of(g * _L, _L), _L)
            cvox_vm[sl] = _TOTAL + jnp.bitwise_and(g + iota, _K - 1)
            cwin_vm[sl] = lax.rem(start + g * _L + iota, n_pts)
            return carry

        lax.fori_loop(0, _CAP // _L, fbody, 0)

        pltpu.sync_copy(winner_hbm.at[pl.ds(start, _LEN_MAX)],
                        scr_vm.at[pl.ds(0, _LEN_MAX)])

        def abody(g, off):
            w = scr_vm[pl.ds(pl.multiple_of(g * _L, _L), _L)]
            mask = w >= 0
            sl = pl.ds(off, _L)
            plsc.store_compressed(cvox_vm.at[sl], start + g * _L + iota,
                                  mask=mask)
            plsc.store_compressed(cwin_vm.at[sl], w, mask=mask)
            return off + jnp.sum(mask.astype(jnp.int32))

        nc = lax.fori_loop(0, ngrp, abody, 0)
        nloop = lax.shift_right_logical(nc + (_K - 1), 6)

        ncb = lax.shift_right_logical(nc + (_CB - 1), 7)

        def cfire(j, carry):
            sl = pl.ds(pl.multiple_of(j * _CB, _CB), _CB)
            pltpu.async_copy(crd_hbm.at[cwin_vm.at[sl]], scr_vm.at[sl], csem)
            return carry

        lax.fori_loop(0, ncb, cfire, 0)

        def cdrain(j, carry):
            sl = pl.ds(pl.multiple_of(j * _CB, _CB), _CB)
            pltpu.make_async_copy(crd_hbm.at[cwin_vm.at[sl]],
                                  scr_vm.at[sl], csem).wait()
            return carry

        lax.fori_loop(0, ncb, cdrain, 0)

        def prep(ic, b):
            @pl.when(ic >= 2)
            def _():
                pltpu.make_async_copy(out_vm.at[b],
                                      out_hbm.at[voxc_vm.at[b]],
                                      ssem[b]).wait()
            cb = pl.multiple_of(ic * _K, _K)
            for g in range(_NG):
                sl = pl.ds(g * _L, _L)
                voxc_vm[b, sl] = cvox_vm[pl.ds(cb + g * _L, _L)]
                real = (cb + g * _L + iota) < nc
                crd = scr_vm[pl.ds(cb + g * _L, _L)]
                yf = lax.shift_right_logical(crd, 9).astype(jnp.float32)
                xf = jnp.bitwise_and(crd, 511).astype(jnp.float32)
                y0 = (yf - 2.0) * 0.25
                x0 = (xf - 2.0) * 0.25
                valid = ((y0 >= -1.0) & (y0 <= float(_H))
                         & (x0 >= -1.0) & (x0 <= float(_W)))
                keep = valid & real
                y = jnp.maximum(y0, 0.0)
                x = jnp.maximum(x0, 0.0)
                yl = y.astype(jnp.int32)
                xl = x.astype(jnp.int32)
                ly = jnp.where(yl >= _H - 1, 0.0, y - yl.astype(jnp.float32))
                lx = jnp.where(xl >= _W - 1, 0.0, x - xl.astype(jnp.float32))
                yl = jnp.minimum(yl, _H - 1)
                xl = jnp.minimum(xl, _W - 1)
                yh = jnp.minimum(yl + 1, _H - 1)
                xh = jnp.minimum(xl + 1, _W - 1)
                scale = jnp.where(keep, 1.0, 0.0)
                hy = (1.0 - ly) * scale
                lys = ly * scale
                hx = 1.0 - lx
                p_vm[b, 0, sl] = yl * _W + xl
                p_vm[b, 1, sl] = yl * _W + xh
                p_vm[b, 2, sl] = yh * _W + xl
                p_vm[b, 3, sl] = yh * _W + xh
                w_vm[b, 0, sl] = hy * hx
                w_vm[b, 1, sl] = hy * lx
                w_vm[b, 2, sl] = lys * hx
                w_vm[b, 3, sl] = lys * lx
            for t in range(4):
                pltpu.async_copy(table_hbm.at[p_vm.at[b, t]],
                                 gbufs[t].at[b], gsem[b])

        def drain_gathers(b):
            for t in range(4):
                pltpu.make_async_copy(table_hbm.at[p_vm.at[b, t]],
                                      gbufs[t].at[b], gsem[b]).wait()

        def blend(b):
            for g in range(_NG):

                def vbody(j, carry2, g=g):
                    jj = zeros + (g * _L) + j
                    b0 = plsc.load_gather(w_vm.at[b, 0], [jj])
                    b1 = plsc.load_gather(w_vm.at[b, 1], [jj])
                    b2 = plsc.load_gather(w_vm.at[b, 2], [jj])
                    b3 = plsc.load_gather(w_vm.at[b, 3], [jj])
                    v = g * _L + j
                    for cg in range(_CG):
                        cs = pl.ds(cg * _L, _L)
                        out_vm[b, v, cs] = (b0 * g0_vm[b, v, cs]
                                            + b1 * g1_vm[b, v, cs]
                                            + b2 * g2_vm[b, v, cs]
                                            + b3 * g3_vm[b, v, cs])
                    return carry2

                lax.fori_loop(0, _L, vbody, 0)

        def scatter(b):
            pltpu.async_copy(out_vm.at[b], out_hbm.at[voxc_vm.at[b]], ssem[b])

        @pl.when(nloop > 0)
        def _():
            prep(0, 0)

        def pair_body(t, carry):
            for b in range(2):
                ic = 2 * t + b

                @pl.when(ic < nloop)
                def _(ic=ic, b=b):
                    @pl.when(ic + 1 < nloop)
                    def _():
                        prep(ic + 1, 1 - b)

                    drain_gathers(b)
                    blend(b)
                    scatter(b)

            return carry

        lax.fori_loop(0, lax.shift_right_logical(nloop + 1, 1), pair_body, 0)

        @pl.when(nloop >= 1)
        def _():
            pltpu.make_async_copy(out_vm.at[0], out_hbm.at[voxc_vm.at[0]],
                                  ssem[0]).wait()

        @pl.when(nloop >= 2)
        def _():
            pltpu.make_async_copy(out_vm.at[1], out_hbm.at[voxc_vm.at[1]],
                                  ssem[1]).wait()

    return k(table, winner, coords)


def kernel(x2d, voxel_indices, img_indices, dist_to_cam):
    del dist_to_cam
    table = jnp.transpose(x2d, (1, 2, 0)).reshape(_HW, _C)
    n = voxel_indices.shape[0]
    flat = (voxel_indices[:, 0] * (_SCENE[1] * _SCENE[2])
            + voxel_indices[:, 1] * _SCENE[2]
            + voxel_indices[:, 2]).astype(jnp.int32)
    winner = jnp.full((_TOTAL,), -1, jnp.int32).at[flat].max(
        jnp.arange(n, dtype=jnp.int32))
    img = img_indices.astype(jnp.int32)
    coords = img[:, 0] * 512 + img[:, 1]
    out = _sc_droi(table, winner, coords)
    res = jnp.where(winner[None, :] >= 0, jnp.transpose(out[:_TOTAL]), 0.0)
    return res.reshape(_C, *_SCENE)
